# baseline (speedup 1.0000x reference)
import functools

import jax
import jax.numpy as jnp
from jax import lax
from jax.experimental import pallas as pl
from jax.experimental.pallas import tpu as pltpu
from jax.experimental.pallas import tpu_sc as plsc

_F32 = jnp.float32
_I32 = jnp.int32

_NW = 16
_D = 2048
_LM = 128
_MAX_LM = 64
_B = 4
_K = 64
_NTOK = _B * 4096
_ROWS = _B * _K
_RPW = _ROWS // _NW
_TPW = _NTOK // _NW
_CPW = _D // _NW
_OPW = _LM // _NW
_AEPW = (_MAX_LM * _LM) // _NW


def _sqrt_pos(x):
    i = lax.bitcast_convert_type(x, _I32)
    y = lax.bitcast_convert_type(jnp.int32(0x5F3759DF) - (i >> 1), _F32)
    for _ in range(3):
        y = y * (1.5 - 0.5 * x * y * y)
    return x * y


def _splat(vec, lane, i):
    return jnp.full((16,), jnp.sum(jnp.where(lane == i, vec, _F32(0.0))), _F32)


def _body(x_h, ttt_h, meta_h, sgr_h, w_h, aein_h, aiin_h, narr_h,
          aeout_h, aiout_h, lmout_h, scal_h,
          rows_v, tch, sgr_v, sgr64_v, imp64_v, acc16, sums_v, coef,
          part, red_v, rawc, raw_v, wv, out8, lm16, old16, new16, ai_v,
          meta_v, scal_v, arch_v, narr_v,
          sem_rows, sem_imp, sem_w, sem_arch, sem_ttt, sem_small,
          spm_sums, spm_parts, spm_raw, spm_lm):
    w = lax.axis_index("s")
    lane = lax.iota(_I32, 16)
    b = w // 4
    q = w % 4

    pltpu.sync_copy(sgr_h.at[pl.ds(w * _RPW, _RPW)], sgr_v)
    pltpu.sync_copy(sgr_h.at[pl.ds(b * _K, _K)], sgr64_v)
    cp_rows = pltpu.async_copy(x_h.at[sgr_v], rows_v, sem_rows)
    cp_imp = pltpu.async_copy(ttt_h.at[sgr64_v], imp64_v, sem_imp)
    cp_w = pltpu.async_copy(w_h.at[pl.ds(w * _OPW, _OPW)], wv, sem_w)
    cp_arch = pltpu.async_copy(aein_h.at[pl.ds(w * _AEPW, _AEPW)], arch_v, sem_arch)
    cp_ttt = pltpu.async_copy(ttt_h.at[pl.ds(w * _TPW, _TPW)], tch, sem_ttt)
    cp_meta = pltpu.async_copy(meta_h, meta_v, sem_small)

    cp_ttt.wait()

    def _tsum(i, a):
        for u in range(4):
            a = a + tch[pl.ds((i * 4 + u) * 16, 16)]
        return a
    acc = lax.fori_loop(0, _TPW // 64, _tsum, jnp.zeros((16,), _F32))
    acc16[...] = acc
    pltpu.sync_copy(acc16, spm_sums.at[pl.ds(w * 16, 16)])

    cp_imp.wait()
    iv = [imp64_v[16 * i:16 * i + 16] for i in range(4)]
    mx = jnp.maximum(jnp.maximum(iv[0], iv[1]), jnp.maximum(iv[2], iv[3]))
    mval = jnp.max(mx)
    es = [jnp.exp(_F32(5.0) * (x - mval)) for x in iv]
    den = jnp.sum(es[0] + es[1] + es[2] + es[3])
    sel = es[0] * (q == 0).astype(_F32)
    for i in range(1, 4):
        sel = sel + es[i] * (q == i).astype(_F32)
    coef[...] = sel * (_F32(0.25) / jnp.full((16,), den, _F32))

    coefv = coef[...]
    cs = [_splat(coefv, lane, i) for i in range(_RPW)]
    cp_rows.wait()

    def _wsum(g4, _):
        for u in range(4):
            g = g4 * 4 + u
            a = cs[0] * rows_v[0, pl.ds(g * 16, 16)]
            for i in range(1, _RPW):
                a = a + cs[i] * rows_v[i, pl.ds(g * 16, 16)]
            part[pl.ds(g * 16, 16)] = a
        return 0
    lax.fori_loop(0, _D // 64, _wsum, 0)
    pltpu.sync_copy(part, spm_parts.at[w])
    plsc.subcore_barrier()

    pltpu.sync_copy(spm_sums, sums_v)
    tot = sums_v[0:16]
    for i in range(1, _NW):
        tot = tot + sums_v[16 * i:16 * i + 16]
    mean_err = jnp.sum(tot) * _F32(1.0 / _NTOK)

    cp_meta.wait()
    tpv = meta_v[0:16]
    tmask = jnp.logical_and(lane % 3 == 2, lane < 12)
    full_prob = jnp.sum(jnp.where(tmask, tpv, _F32(0.0))) * _F32(0.25)
    emav = meta_v[16:32]
    m = jnp.sum(jnp.where(lane == 0, emav, _F32(0.0)))
    v = jnp.sum(jnp.where(lane == 1, emav, _F32(0.0)))
    delta = mean_err - m
    new_mean = m + _F32(0.05) * delta
    new_var = v * _F32(0.95) + _F32(0.05) * delta * delta
    thr = jnp.maximum(new_mean + _F32(0.5) * _sqrt_pos(jnp.maximum(new_var, _F32(1e-8))),
                      _F32(0.3))
    skip = jnp.logical_and(mean_err < thr, full_prob < _F32(0.5))
    af = jnp.where(skip, _F32(0.0), _F32(1.0))
    score = mean_err * full_prob + _F32(1e-6)


    pltpu.sync_copy(spm_parts.at[:, pl.ds(w * _CPW, _CPW)], red_v)
    for g in range(_CPW // 16):
        a = red_v[0, 16 * g:16 * g + 16]
        for i in range(1, _NW):
            a = a + red_v[i, 16 * g:16 * g + 16]
        rawc[16 * g:16 * g + 16] = a
    pltpu.sync_copy(rawc, spm_raw.at[pl.ds(w * _CPW, _CPW)])
    plsc.subcore_barrier()

    cp_w.wait()
    pltpu.sync_copy(spm_raw, raw_v)

    def _dot(g4, accs):
        for u in range(4):
            g = g4 * 4 + u
            r = raw_v[pl.ds(g * 16, 16)]
            accs = tuple(accs[o] + wv[o, pl.ds(g * 16, 16)] * r
                         for o in range(_OPW))
        return accs
    accs = lax.fori_loop(0, _D // 64, _dot,
                         tuple(jnp.zeros((16,), _F32) for _ in range(_OPW)))
    res = jnp.zeros((16,), _F32)
    for o in range(_OPW):
        res = res + jnp.sum(accs[o]) * (lane == o).astype(_F32)
    out8[...] = res
    pltpu.sync_copy(out8.at[pl.ds(0, _OPW)], lmout_h.at[pl.ds(w * _OPW, _OPW)])
    pltpu.sync_copy(out8.at[pl.ds(0, _OPW)], spm_lm.at[pl.ds(w * _OPW, _OPW)])

    cp_arch.wait()
    pltpu.sync_copy(arch_v, aeout_h.at[pl.ds(w * _AEPW, _AEPW)])
    plsc.subcore_barrier()

    pltpu.sync_copy(narr_h, narr_v)
    n = jnp.sum(jnp.where(lane == 0, narr_v[...], jnp.int32(0)))

    @pl.when(w < 8)
    def _():
        pltpu.sync_copy(spm_lm.at[pl.ds(w * 16, 16)], lm16)
        pltpu.sync_copy(aein_h.at[pl.ds(n * _LM + w * 16, 16)], old16)
        new16[...] = lm16[...] * af + old16[...] * (_F32(1.0) - af)
        pltpu.sync_copy(new16, aeout_h.at[pl.ds(n * _LM + w * 16, 16)])

    @pl.when(w == 15)
    def _():
        pltpu.sync_copy(aiin_h, ai_v)
        nsp = jnp.full((16,), n, _I32)
        old = plsc.load_gather(ai_v, [nsp])
        newi = score * af + old * (_F32(1.0) - af)
        plsc.store_scatter(ai_v, [nsp], newi, mask=lane == 0)
        pltpu.sync_copy(ai_v, aiout_h)

    @pl.when(w == 0)
    def _():
        sv = jnp.where(lane == 0, new_mean,
                       jnp.where(lane == 1, new_var,
                                 jnp.where(lane == 2, af, _F32(0.0))))
        scal_v[...] = sv
        pltpu.sync_copy(scal_v, scal_h)


@functools.partial(
    pl.kernel,
    out_type=[
        jax.ShapeDtypeStruct((_MAX_LM * _LM,), _F32),
        jax.ShapeDtypeStruct((_MAX_LM,), _F32),
        jax.ShapeDtypeStruct((_LM,), _F32),
        jax.ShapeDtypeStruct((16,), _F32),
    ],
    mesh=plsc.VectorSubcoreMesh(core_axis_name="c", subcore_axis_name="s",
                                num_cores=1),
    compiler_params=pltpu.CompilerParams(needs_layout_passes=False),
    scratch_types=[
        pltpu.VMEM((_RPW, _D), _F32),
        pltpu.VMEM((_TPW,), _F32),
        pltpu.VMEM((_RPW,), _I32),
        pltpu.VMEM((_K,), _I32),
        pltpu.VMEM((_K,), _F32),
        pltpu.VMEM((16,), _F32),
        pltpu.VMEM((_NW * 16,), _F32),
        pltpu.VMEM((16,), _F32),
        pltpu.VMEM((_D,), _F32),
        pltpu.VMEM((_NW, _CPW), _F32),
        pltpu.VMEM((_CPW,), _F32),
        pltpu.VMEM((_D,), _F32),
        pltpu.VMEM((_OPW, _D), _F32),
        pltpu.VMEM((16,), _F32),
        pltpu.VMEM((16,), _F32),
        pltpu.VMEM((16,), _F32),
        pltpu.VMEM((16,), _F32),
        pltpu.VMEM((_MAX_LM,), _F32),
        pltpu.VMEM((32,), _F32),
        pltpu.VMEM((16,), _F32),
        pltpu.VMEM((_AEPW,), _F32),
        pltpu.VMEM((16,), _I32),
        pltpu.SemaphoreType.DMA,
        pltpu.SemaphoreType.DMA,
        pltpu.SemaphoreType.DMA,
        pltpu.SemaphoreType.DMA,
        pltpu.SemaphoreType.DMA,
        pltpu.SemaphoreType.DMA,
        pltpu.VMEM_SHARED((_NW * 16,), _F32),
        pltpu.VMEM_SHARED((_NW, _D), _F32),
        pltpu.VMEM_SHARED((_D,), _F32),
        pltpu.VMEM_SHARED((_LM,), _F32),
    ],
)
def _landmark_sc(*refs):
    _body(*refs)


def kernel(scan_out, ttt_importance, tier_probs, sgr_indices, W_compress,
           archived_embeddings, archived_importance, err_ema_mean, err_ema_var,
           n_archived):
    x2d = scan_out.reshape(_NTOK, _D)
    ttt1 = ttt_importance.reshape(_NTOK)
    meta = jnp.zeros((32,), _F32)
    meta = meta.at[:12].set(tier_probs.reshape(12).astype(_F32))
    meta = meta.at[16].set(err_ema_mean).at[17].set(err_ema_var)
    base = (jnp.arange(_B, dtype=_I32) * 4096)[:, None]
    sgr = (sgr_indices.astype(_I32) + base).reshape(_ROWS)
    narr = jnp.full((16,), jnp.asarray(n_archived, _I32))

    aeout, aiout, lmout, scal = _landmark_sc(
        x2d, ttt1, meta, sgr, W_compress,
        archived_embeddings.reshape(_MAX_LM * _LM), archived_importance,
        narr)

    return (aeout.reshape(_MAX_LM, _LM), aiout, lmout,
            scal[2] > 0.5, scal[0], scal[1])
